# Initial kernel scaffold; baseline (speedup 1.0000x reference)
import jax
import jax.numpy as jnp
from jax.experimental import pallas as pl


def kernel(x, edge_index, edge_label_index, W1, a1s, a1d, b1, W2, a2s, a2d, b2, W3, a3s, a3d, b3):
    raise NotImplementedError("write your pallas kernel here")


# speedup vs baseline: 15.4859x; 15.4859x over previous
import functools

import jax
import jax.numpy as jnp
from jax import lax
from jax.experimental import pallas as pl
from jax.experimental.pallas import tpu as pltpu
from jax.experimental.pallas import tpu_sc as plsc

N_NODES = 10000
N_EDGES = 320000
N_LABEL = 4096
ROW_BLK = 1000
BE = 128
NB_TILE = 160
NBLK_PAD = 16 * NB_TILE
E_PAD = NBLK_PAD * BE
NCHUNK = 79
NSP_ROWS = NCHUNK * 128


def _mm_split_body(act_ref, w_ref, as_ref, ad_ref, h_ref, aa_ref):
    j = pl.program_id(1)
    h = jnp.dot(act_ref[...], w_ref[0], preferred_element_type=jnp.float32)
    h_ref[0] = h
    asv = jnp.sum(h * as_ref[0], axis=-1)
    adv = jnp.sum(h * ad_ref[0], axis=-1)
    partial = jnp.stack([asv, adv], axis=1)

    @pl.when(j == 0)
    def _():
        aa_ref[...] = partial

    @pl.when(j != 0)
    def _():
        aa_ref[...] += partial


def _mm_split(act, W, a_s, a_d):
    n, din = act.shape
    grid = (n // ROW_BLK, 4)
    W4 = W.reshape(din, 4, 64).transpose(1, 0, 2)
    return pl.pallas_call(
        _mm_split_body,
        grid=grid,
        in_specs=[
            pl.BlockSpec((ROW_BLK, din), lambda i, j: (i, 0)),
            pl.BlockSpec((1, din, 64), lambda i, j: (j, 0, 0)),
            pl.BlockSpec((1, 1, 64), lambda i, j: (j, 0, 0)),
            pl.BlockSpec((1, 1, 64), lambda i, j: (j, 0, 0)),
        ],
        out_specs=[
            pl.BlockSpec((1, ROW_BLK, 64), lambda i, j: (j, i, 0)),
            pl.BlockSpec((ROW_BLK, 2), lambda i, j: (i, 0)),
        ],
        out_shape=[
            jax.ShapeDtypeStruct((4, n, 64), jnp.float32),
            jax.ShapeDtypeStruct((n, 2), jnp.float32),
        ],
    )(act, W4, a_s.reshape(4, 1, 64), a_d.reshape(4, 1, 64))


def _mm_out_body(act_ref, w_ref, as_ref, ad_ref, h_ref, aa_ref):
    h = jnp.dot(act_ref[...], w_ref[...], preferred_element_type=jnp.float32)
    h_ref[...] = h
    asv = jnp.sum(h * as_ref[...], axis=-1)
    adv = jnp.sum(h * ad_ref[...], axis=-1)
    aa_ref[...] = jnp.stack([asv, adv], axis=1)


def _mm_out(act, W, a_s, a_d):
    n, din = act.shape
    dout = W.shape[1]
    return pl.pallas_call(
        _mm_out_body,
        grid=(n // ROW_BLK,),
        in_specs=[
            pl.BlockSpec((ROW_BLK, din), lambda i: (i, 0)),
            pl.BlockSpec((din, dout), lambda i: (0, 0)),
            pl.BlockSpec((1, dout), lambda i: (0, 0)),
            pl.BlockSpec((1, dout), lambda i: (0, 0)),
        ],
        out_specs=[
            pl.BlockSpec((ROW_BLK, dout), lambda i: (i, 0)),
            pl.BlockSpec((ROW_BLK, 2), lambda i: (i, 0)),
        ],
        out_shape=[
            jax.ShapeDtypeStruct((n, dout), jnp.float32),
            jax.ShapeDtypeStruct((n, 2), jnp.float32),
        ],
    )(act, W, a_s.reshape(1, dout), a_d.reshape(1, dout))


def _prep_body(aa_ref, t_ref):
    asv = aa_ref[:, 0]
    adv = aa_ref[:, 1]
    m = jnp.max(asv)
    t_ref[...] = jnp.stack([asv, adv, jnp.full_like(asv, m)], axis=0)


def _prep(aa):
    n = aa.shape[0]
    return pl.pallas_call(
        _prep_body,
        out_shape=jax.ShapeDtypeStruct((3, n), jnp.float32),
    )(aa)


def _norm_cat_body(m_ref, d_ref, b_ref, o_ref):
    den = jnp.sum(d_ref[...], axis=1) * 0.5 + 1e-16
    m = jnp.concatenate([m_ref[0], m_ref[1], m_ref[2], m_ref[3]], axis=-1)
    o_ref[...] = jnp.maximum(m / den[:, None] + b_ref[...], 0.0)


def _norm_cat(msgP, denPT, b):
    n = msgP.shape[1]
    return pl.pallas_call(
        _norm_cat_body,
        grid=(n // ROW_BLK,),
        in_specs=[
            pl.BlockSpec((4, ROW_BLK, 64), lambda i: (0, i, 0)),
            pl.BlockSpec((ROW_BLK, 32), lambda i: (i, 0)),
            pl.BlockSpec((1, 256), lambda i: (0, 0)),
        ],
        out_specs=pl.BlockSpec((ROW_BLK, 256), lambda i: (i, 0)),
        out_shape=jax.ShapeDtypeStruct((n, 256), jnp.float32),
    )(msgP, denPT, b.reshape(1, 256))


def _emb_body(m_ref, d_ref, b_ref, o_ref):
    den = jnp.sum(d_ref[...], axis=1) + 2e-16
    m = m_ref[0] + m_ref[1]
    o_ref[...] = m / den[:, None] + b_ref[...]


def _emb_norm(msgP, denPT, b):
    n = msgP.shape[1]
    return pl.pallas_call(
        _emb_body,
        grid=(n // ROW_BLK,),
        in_specs=[
            pl.BlockSpec((2, ROW_BLK, 16), lambda i: (0, i, 0)),
            pl.BlockSpec((ROW_BLK, 32), lambda i: (i, 0)),
            pl.BlockSpec((1, 16), lambda i: (0, 0)),
        ],
        out_specs=pl.BlockSpec((ROW_BLK, 16), lambda i: (i, 0)),
        out_shape=jax.ShapeDtypeStruct((n, 16), jnp.float32),
    )(msgP, denPT, b.reshape(1, 16))


def _sc_edge_body(DH, NQ, NP, hT, t3F, srcF, dstM, msg_out, den_out,
                  srcb, dstb, asv_v, adv_v, m16, den_v, wv, r0, r1,
                  msg_sp, sem0, sem1):
    c = lax.axis_index("c")
    s = lax.axis_index("s")
    wid = c * 16 + s
    lane = lax.iota(jnp.int32, 16)
    zf = jnp.zeros((16,), jnp.float32)
    ngrp = DH // 16

    pltpu.sync_copy(srcF.at[pl.ds(s * (NB_TILE * BE), NB_TILE * BE)], srcb)
    pltpu.sync_copy(dstM.at[pl.ds(s * NB_TILE, NB_TILE)], dstb)
    pltpu.sync_copy(t3F.at[pl.ds(0, N_NODES)], asv_v)
    pltpu.sync_copy(t3F.at[pl.ds(N_NODES, N_NODES)], adv_v)
    pltpu.sync_copy(t3F.at[pl.ds(2 * N_NODES, 16)], m16)

    def _zden(i, _):
        den_v[pl.ds(i * 16, 16)] = zf
        return 0
    lax.fori_loop(0, N_NODES // 16, _zden, 0)

    def _phase_a(k, with_den):
        mv = m16[pl.ds(0, 16)]
        ebase = (s * NB_TILE + k) * BE
        kf = jnp.full((16,), k, jnp.int32)
        for i in range(8):
            s16 = srcb[pl.ds(k * BE + i * 16, 16)]
            d16 = plsc.load_gather(dstb, [kf, lane + i * 16])
            a_s = plsc.load_gather(asv_v, [s16])
            a_d = plsc.load_gather(adv_v, [d16])
            cz = a_d + mv
            cc = jnp.maximum(cz, 0.2 * cz)
            e = a_s + a_d
            e = jnp.maximum(e, 0.2 * e)
            w16 = jnp.exp(e - cc)
            eid = ebase + i * 16 + lane
            w16 = jnp.where(eid < N_EDGES, w16, 0.0)
            wv[pl.ds(i * 16, 16)] = w16
            if with_den:
                plsc.addupdate_scatter(den_v, [d16], w16)

    def _mul(r_ref):
        def body(e, _):
            wb = plsc.load_gather(wv, [jnp.full((16,), e, jnp.int32)])
            for t in range(ngrp):
                r_ref[e, pl.ds(t * 16, 16)] = r_ref[e, pl.ds(t * 16, 16)] * wb
            return 0
        lax.fori_loop(0, BE, body, 0)

    for p in range(NP):
        q = c * NP + p
        hTc = hT.at[q if NQ > 1 else 0]

        def _zrow(e, _):
            for t in range(ngrp):
                r0[e, pl.ds(t * 16, 16)] = zf
            return 0
        lax.fori_loop(0, BE, _zrow, 0)
        for t in range(5):
            chunk = s * 5 + t

            @pl.when(chunk < NCHUNK)
            def _():
                pltpu.sync_copy(r0, msg_sp.at[pl.ds(chunk * 128, 128)])
        plsc.subcore_barrier()

        def _gather(k, r_ref, sem):
            return pltpu.async_copy(
                hTc.at[srcb.at[pl.ds(k * BE, BE)]], r_ref, sem)

        _gather(0, r0, sem0)
        _gather(1, r1, sem1)

        def _step(k2, _):
            for (off, r_ref, sem) in ((0, r0, sem0), (1, r1, sem1)):
                k = k2 + off
                _phase_a(k, p == 0)
                pltpu.make_async_copy(hTc.at[pl.ds(0, BE)], r_ref, sem).wait()
                _mul(r_ref)
                pltpu.sync_copy(r_ref, msg_sp.at[dstb.at[k]], add=True)

                @pl.when(k + 2 < NB_TILE)
                def _():
                    _gather(k + 2, r_ref, sem)
            return 0

        lax.fori_loop(0, NB_TILE // 2, lambda i, a: _step(i * 2, a), 0)

        plsc.subcore_barrier()

        @pl.when(s < 15)
        def _():
            pltpu.sync_copy(msg_sp.at[pl.ds(s * 640, 640)],
                            msg_out.at[q].at[pl.ds(s * 640, 640)])

        @pl.when(s == 15)
        def _():
            pltpu.sync_copy(msg_sp.at[pl.ds(9600, 400)],
                            msg_out.at[q].at[pl.ds(9600, 400)])

    pltpu.sync_copy(den_v, den_out.at[pl.ds(wid * N_NODES, N_NODES)])


@functools.lru_cache(maxsize=None)
def _sc_edge(DH, NQ, NP):
    mesh = plsc.VectorSubcoreMesh(core_axis_name="c", subcore_axis_name="s")
    return pl.kernel(
        functools.partial(_sc_edge_body, DH, NQ, NP),
        mesh=mesh,
        compiler_params=pltpu.CompilerParams(needs_layout_passes=False, use_tc_tiling_on_sc=False),
        out_type=[
            jax.ShapeDtypeStruct((max(NQ, 2), N_NODES, DH), jnp.float32),
            jax.ShapeDtypeStruct((32 * N_NODES,), jnp.float32),
        ],
        scratch_types=[
            pltpu.VMEM((NB_TILE * BE,), jnp.int32),
            pltpu.VMEM((NB_TILE, BE), jnp.int32),
            pltpu.VMEM((N_NODES,), jnp.float32),
            pltpu.VMEM((N_NODES,), jnp.float32),
            pltpu.VMEM((16,), jnp.float32),
            pltpu.VMEM((N_NODES,), jnp.float32),
            pltpu.VMEM((BE,), jnp.float32),
            pltpu.VMEM((BE, DH), jnp.float32),
            pltpu.VMEM((BE, DH), jnp.float32),
            pltpu.VMEM_SHARED((NSP_ROWS, DH), jnp.float32),
            pltpu.SemaphoreType.DMA,
            pltpu.SemaphoreType.DMA,
        ],
    )


def _sc_score_body(emb, lsF, ldF, preds,
                   lsb, ldb, rd, rl, sv, pv, s_sp, sem0):
    c = lax.axis_index("c")
    s = lax.axis_index("s")
    wid = c * 16 + s

    pltpu.sync_copy(ldF.at[pl.ds(s * 256, 256)], ldb)
    acc = jnp.zeros((16,), jnp.float32)
    for half in range(2):
        pltpu.async_copy(emb.at[ldb.at[pl.ds(half * 128, 128)]], rd,
                         sem0).wait()

        def _red(e, a):
            return a + rd[e, pl.ds(0, 16)]
        acc = lax.fori_loop(0, 128, _red, acc)
    sv[pl.ds(0, 16)] = acc
    pltpu.sync_copy(sv, s_sp.at[s])
    plsc.subcore_barrier()
    pltpu.sync_copy(s_sp, rd.at[pl.ds(0, 16)])
    stot = jnp.zeros((16,), jnp.float32)
    for t in range(16):
        stot = stot + rd[t, pl.ds(0, 16)]

    pltpu.sync_copy(lsF.at[pl.ds(wid * 128, 128)], lsb)
    pltpu.async_copy(emb.at[lsb], rl, sem0).wait()
    zf = jnp.zeros((16,), jnp.float32)
    for j in range(8):
        pv[pl.ds(j * 16, 16)] = zf

    def _dot(e, _):
        prod = rl[e, pl.ds(0, 16)] * stot
        plsc.addupdate_scatter(pv, [jnp.full((16,), e, jnp.int32)], prod)
        return 0
    lax.fori_loop(0, 128, _dot, 0)
    pltpu.sync_copy(pv, preds.at[pl.ds(wid * 128, 128)])


@functools.lru_cache(maxsize=None)
def _sc_score():
    return pl.kernel(
        _sc_score_body,
        mesh=plsc.VectorSubcoreMesh(core_axis_name="c", subcore_axis_name="s"),
        compiler_params=pltpu.CompilerParams(needs_layout_passes=False, use_tc_tiling_on_sc=False),
        out_type=jax.ShapeDtypeStruct((N_LABEL,), jnp.float32),
        scratch_types=[
            pltpu.VMEM((128,), jnp.int32),
            pltpu.VMEM((256,), jnp.int32),
            pltpu.VMEM((128, 16), jnp.float32),
            pltpu.VMEM((128, 16), jnp.float32),
            pltpu.VMEM((16,), jnp.float32),
            pltpu.VMEM((128,), jnp.float32),
            pltpu.VMEM_SHARED((16, 16), jnp.float32),
            pltpu.SemaphoreType.DMA,
        ],
    )


def _pad_edges(v):
    return jnp.pad(v, (0, E_PAD - N_EDGES))


def kernel(x, edge_index, edge_label_index,
           W1, a1s, a1d, b1, W2, a2s, a2d, b2, W3, a3s, a3d, b3):
    srcF = _pad_edges(edge_index[0])
    dstF = _pad_edges(edge_index[1])
    dstM = dstF.reshape(NBLK_PAD, BE)

    hT1, aa1 = _mm_split(x, W1, a1s, a1d)
    msgP1, denP1 = _sc_edge(64, 4, 2)(hT1, _prep(aa1).reshape(-1), srcF, dstM)
    act2 = _norm_cat(msgP1, denP1.reshape(32, N_NODES).T, b1)

    hT2, aa2 = _mm_split(act2, W2, a2s, a2d)
    msgP2, denP2 = _sc_edge(64, 4, 2)(hT2, _prep(aa2).reshape(-1), srcF, dstM)
    act3 = _norm_cat(msgP2, denP2.reshape(32, N_NODES).T, b2)

    h3, aa3 = _mm_out(act3, W3, a3s, a3d)
    msgP3, denP3 = _sc_edge(16, 1, 1)(h3.reshape(1, N_NODES, 16),
                                   _prep(aa3).reshape(-1), srcF, dstM)
    emb = _emb_norm(msgP3, denP3.reshape(32, N_NODES).T, b3)

    preds = _sc_score()(emb, edge_label_index[0], edge_label_index[1])
    return preds, emb
